# baseline (speedup 1.0000x reference)
import functools

import jax
import jax.numpy as jnp
from jax import lax
from jax.experimental import pallas as pl
from jax.experimental.pallas import tpu as pltpu
from jax.experimental.pallas import tpu_sc as plsc

VOCAB = 1000000
B = 4096
S = 200
D = 32
C = 20
CP = 32

NC = 2
NS = 16
NW = NC * NS
BPW = B // NW
G = 4
CHUNK = 100
RPG = G * S // CHUNK
NG = BPW // G
INV_S = 1.0 / S

SLAB = 25600
SUB = SLAB // 4
PGRID = -(-VOCAB // SLAB)
PROWS = PGRID * SUB
PV = PROWS * 4


def _proj_body(tt_ref, w_ref, b_ref, o_ref):
    ds = [
        jnp.dot(
            tt_ref[:, pl.ds(q * SUB, SUB)].T,
            w_ref[:, pl.ds(q * 4 * CP, 4 * CP)],
            preferred_element_type=jnp.float32,
        )
        for q in range(4)
    ]
    o_ref[...] = (((ds[0] + ds[1]) + (ds[2] + ds[3])) + b_ref[...]).astype(jnp.bfloat16)


def _tc_project(tt, w32t, b32):
    return pl.pallas_call(
        _proj_body,
        grid=(PGRID,),
        compiler_params=pltpu.CompilerParams(
            fuse_transposed_lhs_in_matmul=True,
        ),
        in_specs=[
            pl.BlockSpec((D, SLAB), lambda i: (0, i)),
            pl.BlockSpec((D, 16 * CP), lambda i: (0, 0)),
            pl.BlockSpec((1, 4 * CP), lambda i: (0, 0)),
        ],
        out_specs=pl.BlockSpec((SUB, 4 * CP), lambda i: (i, 0)),
        out_shape=jax.ShapeDtypeStruct((PROWS, 4 * CP), jnp.bfloat16),
    )(tt, w32t, b32)


def _sc_pooled(x4, p32):
    mesh = plsc.VectorSubcoreMesh(core_axis_name="c", subcore_axis_name="s")

    @functools.partial(
        pl.kernel,
        out_type=jax.ShapeDtypeStruct((B, CP), jnp.float32),
        mesh=mesh,
        compiler_params=pltpu.CompilerParams(
            use_tc_tiling_on_sc=False, needs_layout_passes=False,
        ),
        scratch_types=[
            pltpu.VMEM((NG, RPG * CHUNK), jnp.int32),
            pltpu.VMEM((RPG * CHUNK, CP), jnp.bfloat16),
            pltpu.VMEM((RPG * CHUNK, CP), jnp.bfloat16),
            pltpu.VMEM((BPW, CP), jnp.float32),
            pltpu.SemaphoreType.DMA,
            pltpu.SemaphoreType.DMA,
        ],
    )
    def k(x_hbm, tbl_hbm, out_hbm, idx_v, buf_a, buf_b, out_v, sem_a, sem_b):
        wid = lax.axis_index("s") * NC + lax.axis_index("c")
        row0 = wid * BPW
        pltpu.sync_copy(x_hbm.at[wid], idx_v)

        def gather_start(g, buf, sem):
            pltpu.make_async_copy(tbl_hbm.at[idx_v.at[g]], buf, sem).start()

        def gather_wait(g, buf, sem):
            pltpu.make_async_copy(tbl_hbm.at[idx_v.at[g]], buf, sem).wait()

        def reduce_group(buf, orow0):
            def body(s, accs):
                accs = list(accs)
                for i in range(G):
                    a, bb = plsc.unpack(buf[i * S + s, :], format=plsc.PackFormat.INTERLEAVED)
                    accs[2 * i] = accs[2 * i] + a
                    accs[2 * i + 1] = accs[2 * i + 1] + bb
                return tuple(accs)

            zero = jnp.zeros((16,), jnp.float32)
            accs = lax.fori_loop(0, S, body, (zero,) * (2 * G), unroll=4)
            for i in range(G):
                out_v[orow0 + i, pl.ds(0, 16)] = accs[2 * i] * INV_S
                out_v[orow0 + i, pl.ds(16, 16)] = accs[2 * i + 1] * INV_S

        gather_start(0, buf_a, sem_a)
        gather_start(1, buf_b, sem_b)

        def outer(gp, carry):
            g = 2 * gp
            gather_wait(g, buf_a, sem_a)
            reduce_group(buf_a, g * G)

            @pl.when(g + 2 < NG)
            def _():
                gather_start(g + 2, buf_a, sem_a)

            gather_wait(g + 1, buf_b, sem_b)
            reduce_group(buf_b, (g + 1) * G)

            @pl.when(g + 3 < NG)
            def _():
                gather_start(g + 3, buf_b, sem_b)

            return carry

        lax.fori_loop(0, NG // 2, outer, 0)
        pltpu.sync_copy(out_v, out_hbm.at[pl.ds(row0, BPW)])

    return k(x4, p32)


def kernel(x, table, W, b):
    w32t = jnp.pad(W, ((0, CP - C), (0, 0))).T
    eye4 = jnp.eye(4, dtype=jnp.float32)
    w128 = jnp.einsum("dc,pq->dpqc", w32t, eye4).reshape(D, 16 * CP)
    b128 = jnp.tile(jnp.pad(b, (0, CP - C)), 4).reshape(1, 4 * CP)
    p32 = _tc_project(table.T, w128, b128)
    xi = x.astype(jnp.int32)
    blk = xi // SLAB
    rem = xi % SLAB
    xr = (blk * SUB + rem % SUB) * 4 + rem // SUB
    x4 = xr.reshape(NW, NG, RPG * CHUNK)
    pooled = _sc_pooled(x4, p32.reshape(PV, CP))
    perm = [2 * c if c < 16 else 2 * (c - 16) + 1 for c in range(CP)]
    inv = [0] * CP
    for j, p in enumerate(perm):
        inv[p] = j
    return pooled[:, jnp.array(inv[:C], dtype=jnp.int32)]


# speedup vs baseline: 1.8426x; 1.8426x over previous
import functools

import jax
import jax.numpy as jnp
from jax import lax
from jax.experimental import pallas as pl
from jax.experimental.pallas import tpu as pltpu
from jax.experimental.pallas import tpu_sc as plsc

VOCAB = 1000000
B = 4096
S = 200
D = 32
C = 20
CP = 32

NC = 2
NS = 16
NW = NC * NS
BPW = B // NW
G = 4
CHUNK = 100
RPG = G * S // CHUNK
NG = BPW // G
INV_S = 1.0 / S

SLAB = 51200
SUB = SLAB // 4
PGRID = -(-VOCAB // SLAB)
PROWS = PGRID * SUB
PV = PROWS * 4


def _proj_body(tt_ref, w_ref, b_ref, o_ref):
    ds = [
        jnp.dot(
            tt_ref[:, pl.ds(q * SUB, SUB)].T,
            w_ref[:, pl.ds(q * 4 * CP, 4 * CP)],
            preferred_element_type=jnp.float32,
        )
        for q in range(4)
    ]
    o_ref[...] = ((ds[0] + ds[1]) + (ds[2] + ds[3])) + b_ref[...]


def _tc_project(tt, w32t, b32):
    return pl.pallas_call(
        _proj_body,
        grid=(PGRID,),
        compiler_params=pltpu.CompilerParams(
            fuse_transposed_lhs_in_matmul=True,
        ),
        in_specs=[
            pl.BlockSpec((D, SLAB), lambda i: (0, i)),
            pl.BlockSpec((D, 16 * CP), lambda i: (0, 0)),
            pl.BlockSpec((1, 4 * CP), lambda i: (0, 0)),
        ],
        out_specs=pl.BlockSpec((SUB, 4 * CP), lambda i: (i, 0)),
        out_shape=jax.ShapeDtypeStruct((PROWS, 4 * CP), jnp.float32),
    )(tt, w32t, b32)


def _sc_pooled(x4, p32):
    mesh = plsc.VectorSubcoreMesh(core_axis_name="c", subcore_axis_name="s")

    @functools.partial(
        pl.kernel,
        out_type=jax.ShapeDtypeStruct((B, CP), jnp.float32),
        mesh=mesh,
        compiler_params=pltpu.CompilerParams(use_tc_tiling_on_sc=False),
        scratch_types=[
            pltpu.VMEM((NG, RPG * CHUNK), jnp.int32),
            pltpu.VMEM((RPG * CHUNK, CP), jnp.float32),
            pltpu.VMEM((RPG * CHUNK, CP), jnp.float32),
            pltpu.VMEM((BPW, CP), jnp.float32),
            pltpu.SemaphoreType.DMA,
            pltpu.SemaphoreType.DMA,
        ],
    )
    def k(x_hbm, tbl_hbm, out_hbm, idx_v, buf_a, buf_b, out_v, sem_a, sem_b):
        wid = lax.axis_index("s") * NC + lax.axis_index("c")
        row0 = wid * BPW
        pltpu.sync_copy(x_hbm.at[wid], idx_v)

        def gather_start(g, buf, sem):
            pltpu.make_async_copy(tbl_hbm.at[idx_v.at[g]], buf, sem).start()

        def gather_wait(g, buf, sem):
            pltpu.make_async_copy(tbl_hbm.at[idx_v.at[g]], buf, sem).wait()

        def reduce_group(buf, orow0):
            def body(s, accs):
                accs = list(accs)
                for i in range(G):
                    accs[2 * i] = accs[2 * i] + buf[i * S + s, pl.ds(0, 16)]
                    accs[2 * i + 1] = accs[2 * i + 1] + buf[i * S + s, pl.ds(16, 16)]
                return tuple(accs)

            zero = jnp.zeros((16,), jnp.float32)
            accs = lax.fori_loop(0, S, body, (zero,) * (2 * G), unroll=4)
            for i in range(G):
                out_v[orow0 + i, pl.ds(0, 16)] = accs[2 * i] * INV_S
                out_v[orow0 + i, pl.ds(16, 16)] = accs[2 * i + 1] * INV_S

        gather_start(0, buf_a, sem_a)
        gather_start(1, buf_b, sem_b)

        def outer(gp, carry):
            g = 2 * gp
            gather_wait(g, buf_a, sem_a)
            reduce_group(buf_a, g * G)

            @pl.when(g + 2 < NG)
            def _():
                gather_start(g + 2, buf_a, sem_a)

            gather_wait(g + 1, buf_b, sem_b)
            reduce_group(buf_b, (g + 1) * G)

            @pl.when(g + 3 < NG)
            def _():
                gather_start(g + 3, buf_b, sem_b)

            return carry

        lax.fori_loop(0, NG // 2, outer, 0)
        pltpu.sync_copy(out_v, out_hbm.at[pl.ds(row0, BPW)])

    return k(x4, p32)


def kernel(x, table, W, b):
    w32t = jnp.pad(W, ((0, CP - C), (0, 0))).T
    eye4 = jnp.eye(4, dtype=jnp.float32)
    w128 = jnp.einsum("dc,pq->dpqc", w32t, eye4).reshape(D, 16 * CP)
    b128 = jnp.tile(jnp.pad(b, (0, CP - C)), 4).reshape(1, 4 * CP)
    p32 = _tc_project(table.T, w128, b128)
    xi = x.astype(jnp.int32)
    blk = xi // SLAB
    rem = xi % SLAB
    xr = (blk * SUB + rem % SUB) * 4 + rem // SUB
    x4 = xr.reshape(NW, NG, RPG * CHUNK)
    pooled = _sc_pooled(x4, p32.reshape(PV, CP))
    return pooled[:, :C]
